# baseline (speedup 1.0000x reference)
import jax
import jax.numpy as jnp
from jax.experimental import pallas as pl

_N = 1024 * 1024
_INV_T = 10.0
_GRID = 1


def _body(w_ref, z_ref, o_ref):
    x = (w_ref[...] - z_ref[...]) * _INV_T
    o_ref[...] = jax.nn.sigmoid(x)


def kernel(weights, noises):
    blk = _N // _GRID
    out = pl.pallas_call(
        _body,
        grid=(_GRID,),
        in_specs=[
            pl.BlockSpec((blk,), lambda i: (i,)),
            pl.BlockSpec((blk,), lambda i: (i,)),
        ],
        out_specs=pl.BlockSpec((blk,), lambda i: (i,)),
        out_shape=jax.ShapeDtypeStruct((_N,), jnp.float32),
    )(weights, noises)
    return out


# speedup vs baseline: 1.2150x; 1.2150x over previous
import jax
import jax.numpy as jnp
from jax.experimental import pallas as pl

_N = 1024 * 1024
_INV_T = 10.0
_GRID = 2


def _body(w_ref, z_ref, o_ref):
    x = (w_ref[...] - z_ref[...]) * _INV_T
    o_ref[...] = jax.nn.sigmoid(x)


def kernel(weights, noises):
    blk = _N // _GRID
    out = pl.pallas_call(
        _body,
        grid=(_GRID,),
        in_specs=[
            pl.BlockSpec((blk,), lambda i: (i,)),
            pl.BlockSpec((blk,), lambda i: (i,)),
        ],
        out_specs=pl.BlockSpec((blk,), lambda i: (i,)),
        out_shape=jax.ShapeDtypeStruct((_N,), jnp.float32),
    )(weights, noises)
    return out
